# baseline (speedup 1.0000x reference)
import functools

import jax
import jax.numpy as jnp
from jax import lax
from jax.experimental import pallas as pl
from jax.experimental.pallas import tpu as pltpu
from jax.experimental.pallas import tpu_sc as plsc

N = 10000
E = 160000
NC = 2
NS = 16
CW = 40
NCH = E // (NS * CW)


def _mesh():
    return plsc.VectorSubcoreMesh(core_axis_name="c", subcore_axis_name="s",
                                  num_cores=NC, num_subcores=NS)


def _prop_body(g_hbm, src_hbm, dst_hbm, zero_hbm, u_hbm, src_v, dst_v, rows0,
               rows1, acc, sem0, sem1):
    c = lax.axis_index("c")
    t = lax.axis_index("s")
    @pl.when(t == 0)
    def _():
        pltpu.sync_copy(zero_hbm, acc)

    pltpu.sync_copy(src_hbm.at[t], src_v)
    pltpu.sync_copy(dst_hbm.at[t], dst_v)

    def adj(k, _):
        i = k // (CW // 16)
        j = (k % (CW // 16)) * 16
        src_v[i, pl.ds(j, 16)] = src_v[i, pl.ds(j, 16)] + c * N
        return 0
    lax.fori_loop(0, NCH * (CW // 16), adj, 0, unroll=False)
    plsc.subcore_barrier()

    pltpu.async_copy(g_hbm.at[src_v.at[0]], rows0, sem0)

    def pair(i, _):
        j = 2 * i
        pltpu.make_async_copy(g_hbm.at[src_v.at[j]], rows0, sem0).wait()

        @pl.when(j + 1 < NCH)
        def _():
            pltpu.async_copy(g_hbm.at[src_v.at[j + 1]], rows1, sem1)
        pltpu.sync_copy(rows0, acc.at[dst_v.at[j]], add=True)

        @pl.when(j + 1 < NCH)
        def _():
            pltpu.make_async_copy(g_hbm.at[src_v.at[j + 1]], rows1, sem1).wait()

            @pl.when(j + 2 < NCH)
            def _():
                pltpu.async_copy(g_hbm.at[src_v.at[j + 2]], rows0, sem0)
            pltpu.sync_copy(rows1, acc.at[dst_v.at[j + 1]], add=True)
        return 0
    lax.fori_loop(0, (NCH + 1) // 2, pair, 0, unroll=False)
    plsc.subcore_barrier()

    @pl.when(t == 0)
    def _():
        pltpu.sync_copy(acc, u_hbm.at[c])


@functools.lru_cache(maxsize=None)
def _make_sc_prop():
    return pl.kernel(
        _prop_body,
        out_type=jax.ShapeDtypeStruct((NC, N, 128), jnp.float32),
        mesh=_mesh(),
        compiler_params=pltpu.CompilerParams(use_tc_tiling_on_sc=False),
        scratch_types=[
            pltpu.VMEM((NCH, CW), jnp.int32),
            pltpu.VMEM((NCH, CW), jnp.int32),
            pltpu.VMEM((CW, 128), jnp.float32),
            pltpu.VMEM((CW, 128), jnp.float32),
            pltpu.VMEM_SHARED((N, 128), jnp.float32),
            pltpu.SemaphoreType.DMA,
            pltpu.SemaphoreType.DMA,
        ],
    )


def _sc_prop(g, src3, dst3, zero):
    u = _make_sc_prop()(g, src3, dst3, zero)
    return u.reshape(NC * N, 128)


NB = 1000
NBLK = N // NB


def _prep_kernel(degw_ref, x_ref, disw_ref, dinvw_ref, g_ref):
    deg = degw_ref[...][:, 0:1]
    dis = jnp.where(deg > 0, lax.rsqrt(jnp.maximum(deg, 1e-12)), 0.0)
    disw_ref[...] = jnp.broadcast_to(dis, (NB, 128))
    dinvw_ref[...] = jnp.broadcast_to(dis * dis, (NB, 128))
    g_ref[...] = x_ref[...] * dis


def _tc_prep(degw, x):
    return pl.pallas_call(
        _prep_kernel,
        grid=(NC, NBLK),
        in_specs=[
            pl.BlockSpec((NB, 128), lambda s, i: (i, 0)),
            pl.BlockSpec((NB, 128), lambda s, i: (i, s)),
        ],
        out_specs=[
            pl.BlockSpec((NB, 128), lambda s, i: (i, 0)),
            pl.BlockSpec((NB, 128), lambda s, i: (i, 0)),
            pl.BlockSpec((NB, 128), lambda s, i: (s * NBLK + i, 0)),
        ],
        out_shape=[
            jax.ShapeDtypeStruct((N, 128), jnp.float32),
            jax.ShapeDtypeStruct((N, 128), jnp.float32),
            jax.ShapeDtypeStruct((NC * N, 128), jnp.float32),
        ],
    )(degw, x)


def _rescale_kernel(u_ref, dinvw_ref, g_ref):
    g_ref[...] = u_ref[...] * dinvw_ref[...]


def _tc_rescale(u, dinvw):
    return pl.pallas_call(
        _rescale_kernel,
        grid=(NC, NBLK),
        in_specs=[
            pl.BlockSpec((NB, 128), lambda s, i: (s * NBLK + i, 0)),
            pl.BlockSpec((NB, 128), lambda s, i: (i, 0)),
        ],
        out_specs=pl.BlockSpec((NB, 128), lambda s, i: (s * NBLK + i, 0)),
        out_shape=jax.ShapeDtypeStruct((NC * N, 128), jnp.float32),
    )(u, dinvw)


def _combine_kernel(h0_ref, u1l, u1h, u2l, u2h, u3l, u3h, disw_ref, w_ref, b_ref,
                    h_ref, g_ref):
    dis = disw_ref[...][:, 0:1]
    f32 = jnp.float32
    acc = jnp.dot(h0_ref[...], w_ref[0], preferred_element_type=f32)
    m = jnp.dot(jnp.concatenate([u1l[...], u1h[...]], axis=1), w_ref[1],
                preferred_element_type=f32)
    m += jnp.dot(jnp.concatenate([u2l[...], u2h[...]], axis=1), w_ref[2],
                 preferred_element_type=f32)
    m += jnp.dot(jnp.concatenate([u3l[...], u3h[...]], axis=1), w_ref[3],
                 preferred_element_type=f32)
    acc = acc + m * dis + b_ref[...][None, :]
    h = jnp.maximum(acc, 0.0)
    h_ref[...] = h[:, :]
    g_ref[...] = h[:, :] * dis


def _tc_combine(h0, u1, u2, u3, disw, w, b):
    ulo = pl.BlockSpec((NB, 128), lambda s, i: (i, 0))
    uhi = pl.BlockSpec((NB, 128), lambda s, i: (NBLK + i, 0))
    return pl.pallas_call(
        _combine_kernel,
        grid=(NC, NBLK),
        in_specs=[
            pl.BlockSpec((NB, 256), lambda s, i: (i, 0)),
            ulo, uhi, ulo, uhi, ulo, uhi,
            pl.BlockSpec((NB, 128), lambda s, i: (i, 0)),
            pl.BlockSpec((4, 256, 128), lambda s, i: (0, 0, s)),
            pl.BlockSpec((128,), lambda s, i: (s,)),
        ],
        out_specs=[
            pl.BlockSpec((NB, 128), lambda s, i: (i, s)),
            pl.BlockSpec((NB, 128), lambda s, i: (s * NBLK + i, 0)),
        ],
        out_shape=[
            jax.ShapeDtypeStruct((N, 256), jnp.float32),
            jax.ShapeDtypeStruct((NC * N, 128), jnp.float32),
        ],
    )(h0, u1, u1, u2, u2, u3, u3, disw, w, b)


def _final_kernel(h0_ref, u1l, u1h, u2l, u2h, u3l, u3h, disw_ref, w_ref, b_ref,
                  out_ref):
    f32 = jnp.float32
    acc = jnp.dot(h0_ref[...], w_ref[0], preferred_element_type=f32)
    m = jnp.dot(jnp.concatenate([u1l[...], u1h[...]], axis=1), w_ref[1],
                preferred_element_type=f32)
    m += jnp.dot(jnp.concatenate([u2l[...], u2h[...]], axis=1), w_ref[2],
                 preferred_element_type=f32)
    m += jnp.dot(jnp.concatenate([u3l[...], u3h[...]], axis=1), w_ref[3],
                 preferred_element_type=f32)
    acc = acc + m * disw_ref[...][:, 0:1] + b_ref[...][None, :]
    out_ref[...] = jnp.tanh(acc)


def _tc_final(h0, u1, u2, u3, disw, w, b):
    ulo = pl.BlockSpec((NB, 128), lambda i: (i, 0))
    uhi = pl.BlockSpec((NB, 128), lambda i: (NBLK + i, 0))
    return pl.pallas_call(
        _final_kernel,
        grid=(NBLK,),
        in_specs=[
            pl.BlockSpec((NB, 256), lambda i: (i, 0)),
            ulo, uhi, ulo, uhi, ulo, uhi,
            pl.BlockSpec((NB, 128), lambda i: (i, 0)),
            pl.BlockSpec((4, 256, 64), lambda i: (0, 0, 0)),
            pl.BlockSpec((64,), lambda i: (0,)),
        ],
        out_specs=pl.BlockSpec((NB, 64), lambda i: (i, 0)),
        out_shape=jax.ShapeDtypeStruct((N, 64), jnp.float32),
    )(h0, u1, u1, u2, u2, u3, u3, disw, w, b)


def kernel(x, edge_index, W1, b1, W2, b2, W3, b3):
    src3 = edge_index[0].reshape(NS, NCH, CW)
    dst3 = edge_index[1].reshape(NS, NCH, CW)
    zero = jnp.zeros((N, 128), jnp.float32)
    ones = jnp.ones((NC * N, 128), jnp.float32)

    degw = _sc_prop(ones, src3, dst3, zero)[:N]
    disw, dinvw, g = _tc_prep(degw, x)

    h = x
    for li, (w, b) in enumerate(((W1, b1), (W2, b2), (W3, b3))):
        u1 = _sc_prop(g, src3, dst3, zero)
        g1 = _tc_rescale(u1, dinvw)
        u2 = _sc_prop(g1, src3, dst3, zero)
        g2 = _tc_rescale(u2, dinvw)
        u3 = _sc_prop(g2, src3, dst3, zero)
        if li < 2:
            h, g = _tc_combine(h, u1, u2, u3, disw, w, b)
        else:
            out = _tc_final(h, u1, u2, u3, disw, w, b)
    return out


# speedup vs baseline: 1.7568x; 1.7568x over previous
import functools

import jax
import jax.numpy as jnp
from jax import lax
from jax.experimental import pallas as pl
from jax.experimental.pallas import tpu as pltpu
from jax.experimental.pallas import tpu_sc as plsc

N = 10000
E = 160000
NC = 2
NS = 16
CW = 80
NCH = E // (NS * CW)


def _mesh():
    return plsc.VectorSubcoreMesh(core_axis_name="c", subcore_axis_name="s",
                                  num_cores=NC, num_subcores=NS)


EPT = NS * NCH * CW // NS


def _prop_body(g_hbm, src_hbm, dst_hbm, zero_hbm, u_hbm, src_v, dbuf0, dbuf1,
               rows0, rows1, acc, sem0, sem1, semd0, semd1):
    c = lax.axis_index("c")
    t = lax.axis_index("s")
    @pl.when(t == 0)
    def _():
        pltpu.sync_copy(zero_hbm, acc)

    pltpu.sync_copy(src_hbm.at[pl.ds(t * EPT, EPT)], src_v)

    def adj(k, _):
        src_v[pl.ds(k * 16, 16)] = src_v[pl.ds(k * 16, 16)] + c * N
        return 0
    lax.fori_loop(0, EPT // 16, adj, 0, unroll=False)
    plsc.subcore_barrier()

    def start(j, rows, dbuf, sem, semd):
        pltpu.async_copy(dst_hbm.at[pl.ds(t * EPT + j * CW, CW)], dbuf, semd)
        pltpu.async_copy(g_hbm.at[src_v.at[pl.ds(j * CW, CW)]], rows, sem)

    def finish(j, rows, dbuf, sem, semd):
        pltpu.make_async_copy(dst_hbm.at[pl.ds(t * EPT + j * CW, CW)], dbuf,
                              semd).wait()
        pltpu.make_async_copy(g_hbm.at[src_v.at[pl.ds(j * CW, CW)]], rows,
                              sem).wait()
        pltpu.sync_copy(rows, acc.at[dbuf], add=True)

    start(0, rows0, dbuf0, sem0, semd0)

    def pair(i, _):
        j = 2 * i

        @pl.when(j + 1 < NCH)
        def _():
            start(j + 1, rows1, dbuf1, sem1, semd1)
        finish(j, rows0, dbuf0, sem0, semd0)

        @pl.when(j + 1 < NCH)
        def _():
            @pl.when(j + 2 < NCH)
            def _():
                start(j + 2, rows0, dbuf0, sem0, semd0)
            finish(j + 1, rows1, dbuf1, sem1, semd1)
        return 0
    lax.fori_loop(0, (NCH + 1) // 2, pair, 0, unroll=False)
    plsc.subcore_barrier()

    @pl.when(t == 0)
    def _():
        pltpu.sync_copy(acc, u_hbm.at[c])


@functools.lru_cache(maxsize=None)
def _make_sc_prop():
    return pl.kernel(
        _prop_body,
        out_type=jax.ShapeDtypeStruct((NC, N, 128), jnp.float32),
        mesh=_mesh(),
        scratch_types=[
            pltpu.VMEM((EPT,), jnp.int32),
            pltpu.VMEM((CW,), jnp.int32),
            pltpu.VMEM((CW,), jnp.int32),
            pltpu.VMEM((CW, 128), jnp.float32),
            pltpu.VMEM((CW, 128), jnp.float32),
            pltpu.VMEM_SHARED((N, 128), jnp.float32),
            pltpu.SemaphoreType.DMA,
            pltpu.SemaphoreType.DMA,
            pltpu.SemaphoreType.DMA,
            pltpu.SemaphoreType.DMA,
        ],
    )


def _sc_prop(g, src3, dst3, zero):
    u = _make_sc_prop()(g, src3, dst3, zero)
    return u.reshape(NC * N, 128)


NB = 1000
NBLK = N // NB


def _prep_kernel(degw_ref, x_ref, disw_ref, dinvw_ref, g_ref):
    deg = degw_ref[...][:, 0:1]
    dis = jnp.where(deg > 0, lax.rsqrt(jnp.maximum(deg, 1e-12)), 0.0)
    disw_ref[...] = jnp.broadcast_to(dis, (NB, 128))
    dinvw_ref[...] = jnp.broadcast_to(dis * dis, (NB, 128))
    g_ref[...] = x_ref[...] * dis


def _tc_prep(degw, x):
    return pl.pallas_call(
        _prep_kernel,
        grid=(NC, NBLK),
        in_specs=[
            pl.BlockSpec((NB, 128), lambda s, i: (i, 0)),
            pl.BlockSpec((NB, 128), lambda s, i: (i, s)),
        ],
        out_specs=[
            pl.BlockSpec((NB, 128), lambda s, i: (i, 0)),
            pl.BlockSpec((NB, 128), lambda s, i: (i, 0)),
            pl.BlockSpec((NB, 128), lambda s, i: (s * NBLK + i, 0)),
        ],
        out_shape=[
            jax.ShapeDtypeStruct((N, 128), jnp.float32),
            jax.ShapeDtypeStruct((N, 128), jnp.float32),
            jax.ShapeDtypeStruct((NC * N, 128), jnp.float32),
        ],
    )(degw, x)


def _rescale_kernel(u_ref, dinvw_ref, g_ref):
    g_ref[...] = u_ref[...] * dinvw_ref[...]


def _tc_rescale(u, dinvw):
    return pl.pallas_call(
        _rescale_kernel,
        grid=(NC, NBLK),
        in_specs=[
            pl.BlockSpec((NB, 128), lambda s, i: (s * NBLK + i, 0)),
            pl.BlockSpec((NB, 128), lambda s, i: (i, 0)),
        ],
        out_specs=pl.BlockSpec((NB, 128), lambda s, i: (s * NBLK + i, 0)),
        out_shape=jax.ShapeDtypeStruct((NC * N, 128), jnp.float32),
    )(u, dinvw)


def _combine_kernel(h0_ref, u1l, u1h, u2l, u2h, u3l, u3h, disw_ref, w_ref, b_ref,
                    h_ref, g_ref):
    dis = disw_ref[...][:, 0:1]
    f32 = jnp.float32
    acc = jnp.dot(h0_ref[...], w_ref[0], preferred_element_type=f32)
    m = jnp.dot(jnp.concatenate([u1l[...], u1h[...]], axis=1), w_ref[1],
                preferred_element_type=f32)
    m += jnp.dot(jnp.concatenate([u2l[...], u2h[...]], axis=1), w_ref[2],
                 preferred_element_type=f32)
    m += jnp.dot(jnp.concatenate([u3l[...], u3h[...]], axis=1), w_ref[3],
                 preferred_element_type=f32)
    acc = acc + m * dis + b_ref[...][None, :]
    h = jnp.maximum(acc, 0.0)
    h_ref[...] = h[:, :]
    g_ref[...] = h[:, :] * dis


def _tc_combine(h0, u1, u2, u3, disw, w, b):
    ulo = pl.BlockSpec((NB, 128), lambda s, i: (i, 0))
    uhi = pl.BlockSpec((NB, 128), lambda s, i: (NBLK + i, 0))
    return pl.pallas_call(
        _combine_kernel,
        grid=(NC, NBLK),
        in_specs=[
            pl.BlockSpec((NB, 256), lambda s, i: (i, 0)),
            ulo, uhi, ulo, uhi, ulo, uhi,
            pl.BlockSpec((NB, 128), lambda s, i: (i, 0)),
            pl.BlockSpec((4, 256, 128), lambda s, i: (0, 0, s)),
            pl.BlockSpec((128,), lambda s, i: (s,)),
        ],
        out_specs=[
            pl.BlockSpec((NB, 128), lambda s, i: (i, s)),
            pl.BlockSpec((NB, 128), lambda s, i: (s * NBLK + i, 0)),
        ],
        out_shape=[
            jax.ShapeDtypeStruct((N, 256), jnp.float32),
            jax.ShapeDtypeStruct((NC * N, 128), jnp.float32),
        ],
    )(h0, u1, u1, u2, u2, u3, u3, disw, w, b)


def _final_kernel(h0_ref, u1l, u1h, u2l, u2h, u3l, u3h, disw_ref, w_ref, b_ref,
                  out_ref):
    f32 = jnp.float32
    acc = jnp.dot(h0_ref[...], w_ref[0], preferred_element_type=f32)
    m = jnp.dot(jnp.concatenate([u1l[...], u1h[...]], axis=1), w_ref[1],
                preferred_element_type=f32)
    m += jnp.dot(jnp.concatenate([u2l[...], u2h[...]], axis=1), w_ref[2],
                 preferred_element_type=f32)
    m += jnp.dot(jnp.concatenate([u3l[...], u3h[...]], axis=1), w_ref[3],
                 preferred_element_type=f32)
    acc = acc + m * disw_ref[...][:, 0:1] + b_ref[...][None, :]
    out_ref[...] = jnp.tanh(acc)


def _tc_final(h0, u1, u2, u3, disw, w, b):
    ulo = pl.BlockSpec((NB, 128), lambda i: (i, 0))
    uhi = pl.BlockSpec((NB, 128), lambda i: (NBLK + i, 0))
    return pl.pallas_call(
        _final_kernel,
        grid=(NBLK,),
        in_specs=[
            pl.BlockSpec((NB, 256), lambda i: (i, 0)),
            ulo, uhi, ulo, uhi, ulo, uhi,
            pl.BlockSpec((NB, 128), lambda i: (i, 0)),
            pl.BlockSpec((4, 256, 64), lambda i: (0, 0, 0)),
            pl.BlockSpec((64,), lambda i: (0,)),
        ],
        out_specs=pl.BlockSpec((NB, 64), lambda i: (i, 0)),
        out_shape=jax.ShapeDtypeStruct((N, 64), jnp.float32),
    )(h0, u1, u1, u2, u2, u3, u3, disw, w, b)


def kernel(x, edge_index, W1, b1, W2, b2, W3, b3):
    src3 = edge_index[0]
    dst3 = edge_index[1]
    zero = jnp.zeros((N, 128), jnp.float32)
    ones = jnp.ones((NC * N, 128), jnp.float32)

    degw = _sc_prop(ones, src3, dst3, zero)[:N]
    disw, dinvw, g = _tc_prep(degw, x)

    h = x
    for li, (w, b) in enumerate(((W1, b1), (W2, b2), (W3, b3))):
        u1 = _sc_prop(g, src3, dst3, zero)
        g1 = _tc_rescale(u1, dinvw)
        u2 = _sc_prop(g1, src3, dst3, zero)
        g2 = _tc_rescale(u2, dinvw)
        u3 = _sc_prop(g2, src3, dst3, zero)
        if li < 2:
            h, g = _tc_combine(h, u1, u2, u3, disw, w, b)
        else:
            out = _tc_final(h, u1, u2, u3, disw, w, b)
    return out


# speedup vs baseline: 2.0270x; 1.1538x over previous
import functools

import jax
import jax.numpy as jnp
from jax import lax
from jax.experimental import pallas as pl
from jax.experimental.pallas import tpu as pltpu
from jax.experimental.pallas import tpu_sc as plsc

N = 10000
E = 160000
NC = 2
NS = 16
CW = 40
NCH = E // (NS * CW)


def _mesh():
    return plsc.VectorSubcoreMesh(core_axis_name="c", subcore_axis_name="s",
                                  num_cores=NC, num_subcores=NS)


EPT = NS * NCH * CW // NS


def _prop_body(g_hbm, src_hbm, dst_hbm, zero_hbm, u_hbm, src_v, dbuf0, dbuf1,
               dbuf2, dbuf3, rows0, rows1, rows2, rows3, acc, sem0, sem1, sem2,
               sem3, semd0, semd1, semd2, semd3):
    c = lax.axis_index("c")
    t = lax.axis_index("s")
    @pl.when(t == 0)
    def _():
        pltpu.sync_copy(zero_hbm, acc)

    pltpu.sync_copy(src_hbm.at[pl.ds(t * EPT, EPT)], src_v)

    def adj(k, _):
        src_v[pl.ds(k * 16, 16)] = src_v[pl.ds(k * 16, 16)] + c * N
        return 0
    lax.fori_loop(0, EPT // 16, adj, 0, unroll=False)
    plsc.subcore_barrier()

    BUFS = ((rows0, dbuf0, sem0, semd0), (rows1, dbuf1, sem1, semd1),
            (rows2, dbuf2, sem2, semd2), (rows3, dbuf3, sem3, semd3))
    NBUF = 4

    def start(j, b):
        rows, dbuf, sem, semd = b
        pltpu.async_copy(dst_hbm.at[pl.ds(t * EPT + j * CW, CW)], dbuf, semd)
        pltpu.async_copy(g_hbm.at[src_v.at[pl.ds(j * CW, CW)]], rows, sem)

    def finish(j, b):
        rows, dbuf, sem, semd = b
        pltpu.make_async_copy(dst_hbm.at[pl.ds(t * EPT + j * CW, CW)], dbuf,
                              semd).wait()
        pltpu.make_async_copy(g_hbm.at[src_v.at[pl.ds(j * CW, CW)]], rows,
                              sem).wait()
        pltpu.sync_copy(rows, acc.at[dbuf], add=True)

    for q in range(NBUF - 1):
        start(q, BUFS[q])

    def quad(i, _):
        j = i * NBUF
        for q in range(NBUF):
            jq = j + q

            @pl.when(jq + NBUF - 1 < NCH)
            def _():
                start(jq + NBUF - 1, BUFS[(q + NBUF - 1) % NBUF])
            finish(jq, BUFS[q])
        return 0
    lax.fori_loop(0, NCH // NBUF, quad, 0, unroll=False)
    for r in range(NCH - NBUF * (NCH // NBUF)):
        finish(NBUF * (NCH // NBUF) + r, BUFS[r])
    plsc.subcore_barrier()

    @pl.when(t == 0)
    def _():
        pltpu.sync_copy(acc, u_hbm.at[c])


@functools.lru_cache(maxsize=None)
def _make_sc_prop():
    return pl.kernel(
        _prop_body,
        out_type=jax.ShapeDtypeStruct((NC, N, 128), jnp.float32),
        mesh=_mesh(),
        scratch_types=(
            [pltpu.VMEM((EPT,), jnp.int32)]
            + [pltpu.VMEM((CW,), jnp.int32)] * 4
            + [pltpu.VMEM((CW, 128), jnp.float32)] * 4
            + [pltpu.VMEM_SHARED((N, 128), jnp.float32)]
            + [pltpu.SemaphoreType.DMA] * 8
        ),
    )


def _sc_prop(g, src3, dst3, zero):
    u = _make_sc_prop()(g, src3, dst3, zero)
    return u.reshape(NC * N, 128)


NB = 1000
NBLK = N // NB


def _prep_kernel(degw_ref, x_ref, disw_ref, dinvw_ref, g_ref):
    deg = degw_ref[...][:, 0:1]
    dis = jnp.where(deg > 0, lax.rsqrt(jnp.maximum(deg, 1e-12)), 0.0)
    disw_ref[...] = jnp.broadcast_to(dis, (NB, 128))
    dinvw_ref[...] = jnp.broadcast_to(dis * dis, (NB, 128))
    g_ref[...] = x_ref[...] * dis


def _tc_prep(degw, x):
    return pl.pallas_call(
        _prep_kernel,
        grid=(NC, NBLK),
        in_specs=[
            pl.BlockSpec((NB, 128), lambda s, i: (i, 0)),
            pl.BlockSpec((NB, 128), lambda s, i: (i, s)),
        ],
        out_specs=[
            pl.BlockSpec((NB, 128), lambda s, i: (i, 0)),
            pl.BlockSpec((NB, 128), lambda s, i: (i, 0)),
            pl.BlockSpec((NB, 128), lambda s, i: (s * NBLK + i, 0)),
        ],
        out_shape=[
            jax.ShapeDtypeStruct((N, 128), jnp.float32),
            jax.ShapeDtypeStruct((N, 128), jnp.float32),
            jax.ShapeDtypeStruct((NC * N, 128), jnp.float32),
        ],
    )(degw, x)


def _rescale_kernel(u_ref, dinvw_ref, g_ref):
    g_ref[...] = u_ref[...] * dinvw_ref[...]


def _tc_rescale(u, dinvw):
    return pl.pallas_call(
        _rescale_kernel,
        grid=(NC, NBLK),
        in_specs=[
            pl.BlockSpec((NB, 128), lambda s, i: (s * NBLK + i, 0)),
            pl.BlockSpec((NB, 128), lambda s, i: (i, 0)),
        ],
        out_specs=pl.BlockSpec((NB, 128), lambda s, i: (s * NBLK + i, 0)),
        out_shape=jax.ShapeDtypeStruct((NC * N, 128), jnp.float32),
    )(u, dinvw)


def _combine_kernel(h0_ref, u1l, u1h, u2l, u2h, u3l, u3h, disw_ref, w_ref, b_ref,
                    h_ref, g_ref):
    dis = disw_ref[...][:, 0:1]
    f32 = jnp.float32
    acc = jnp.dot(h0_ref[...], w_ref[0], preferred_element_type=f32)
    m = jnp.dot(jnp.concatenate([u1l[...], u1h[...]], axis=1), w_ref[1],
                preferred_element_type=f32)
    m += jnp.dot(jnp.concatenate([u2l[...], u2h[...]], axis=1), w_ref[2],
                 preferred_element_type=f32)
    m += jnp.dot(jnp.concatenate([u3l[...], u3h[...]], axis=1), w_ref[3],
                 preferred_element_type=f32)
    acc = acc + m * dis + b_ref[...][None, :]
    h = jnp.maximum(acc, 0.0)
    h_ref[...] = h[:, :]
    g_ref[...] = h[:, :] * dis


def _tc_combine(h0, u1, u2, u3, disw, w, b):
    ulo = pl.BlockSpec((NB, 128), lambda s, i: (i, 0))
    uhi = pl.BlockSpec((NB, 128), lambda s, i: (NBLK + i, 0))
    return pl.pallas_call(
        _combine_kernel,
        grid=(NC, NBLK),
        in_specs=[
            pl.BlockSpec((NB, 256), lambda s, i: (i, 0)),
            ulo, uhi, ulo, uhi, ulo, uhi,
            pl.BlockSpec((NB, 128), lambda s, i: (i, 0)),
            pl.BlockSpec((4, 256, 128), lambda s, i: (0, 0, s)),
            pl.BlockSpec((128,), lambda s, i: (s,)),
        ],
        out_specs=[
            pl.BlockSpec((NB, 128), lambda s, i: (i, s)),
            pl.BlockSpec((NB, 128), lambda s, i: (s * NBLK + i, 0)),
        ],
        out_shape=[
            jax.ShapeDtypeStruct((N, 256), jnp.float32),
            jax.ShapeDtypeStruct((NC * N, 128), jnp.float32),
        ],
    )(h0, u1, u1, u2, u2, u3, u3, disw, w, b)


def _final_kernel(h0_ref, u1l, u1h, u2l, u2h, u3l, u3h, disw_ref, w_ref, b_ref,
                  out_ref):
    f32 = jnp.float32
    acc = jnp.dot(h0_ref[...], w_ref[0], preferred_element_type=f32)
    m = jnp.dot(jnp.concatenate([u1l[...], u1h[...]], axis=1), w_ref[1],
                preferred_element_type=f32)
    m += jnp.dot(jnp.concatenate([u2l[...], u2h[...]], axis=1), w_ref[2],
                 preferred_element_type=f32)
    m += jnp.dot(jnp.concatenate([u3l[...], u3h[...]], axis=1), w_ref[3],
                 preferred_element_type=f32)
    acc = acc + m * disw_ref[...][:, 0:1] + b_ref[...][None, :]
    out_ref[...] = jnp.tanh(acc)


def _tc_final(h0, u1, u2, u3, disw, w, b):
    ulo = pl.BlockSpec((NB, 128), lambda i: (i, 0))
    uhi = pl.BlockSpec((NB, 128), lambda i: (NBLK + i, 0))
    return pl.pallas_call(
        _final_kernel,
        grid=(NBLK,),
        in_specs=[
            pl.BlockSpec((NB, 256), lambda i: (i, 0)),
            ulo, uhi, ulo, uhi, ulo, uhi,
            pl.BlockSpec((NB, 128), lambda i: (i, 0)),
            pl.BlockSpec((4, 256, 64), lambda i: (0, 0, 0)),
            pl.BlockSpec((64,), lambda i: (0,)),
        ],
        out_specs=pl.BlockSpec((NB, 64), lambda i: (i, 0)),
        out_shape=jax.ShapeDtypeStruct((N, 64), jnp.float32),
    )(h0, u1, u1, u2, u2, u3, u3, disw, w, b)


def kernel(x, edge_index, W1, b1, W2, b2, W3, b3):
    src3 = edge_index[0]
    dst3 = edge_index[1]
    zero = jnp.zeros((N, 128), jnp.float32)
    ones = jnp.ones((NC * N, 128), jnp.float32)

    degw = _sc_prop(ones, src3, dst3, zero)[:N]
    disw, dinvw, g = _tc_prep(degw, x)

    h = x
    for li, (w, b) in enumerate(((W1, b1), (W2, b2), (W3, b3))):
        u1 = _sc_prop(g, src3, dst3, zero)
        g1 = _tc_rescale(u1, dinvw)
        u2 = _sc_prop(g1, src3, dst3, zero)
        g2 = _tc_rescale(u2, dinvw)
        u3 = _sc_prop(g2, src3, dst3, zero)
        if li < 2:
            h, g = _tc_combine(h, u1, u2, u3, disw, w, b)
        else:
            out = _tc_final(h, u1, u2, u3, disw, w, b)
    return out
